# Initial kernel scaffold; baseline (speedup 1.0000x reference)
import jax
import jax.numpy as jnp
from jax.experimental import pallas as pl

_IMG_H = 100.0
_IMG_W = 200.0


def _decode_kernel(x_ref, o_ref):
    cx = x_ref[0, 0, 0, 0]
    cy = x_ref[0, 1, 0, 0]
    w = x_ref[0, 2, 0, 0]
    h = x_ref[0, 3, 0, 0]
    s = x_ref[0, 4, 0, 0]
    dw = w * 0.5
    dh = h * 0.5
    x1 = jnp.clip(cx - dw, 0.0, _IMG_W)
    y1 = jnp.clip(cy - dh, 0.0, _IMG_H)
    x2 = jnp.clip(cx + dw, 0.0, _IMG_W)
    y2 = jnp.clip(cy + dh, 0.0, _IMG_H)
    col = jax.lax.broadcasted_iota(jnp.int32, (8, 128), 1)
    row = jnp.zeros((8, 128), jnp.float32)
    row = jnp.where(col == 1, x1, row)
    row = jnp.where(col == 2, y1, row)
    row = jnp.where(col == 3, x2, row)
    row = jnp.where(col == 4, y2, row)
    row = jnp.where(col == 5, s, row)
    o_ref[...] = row


def kernel(x):
    out = pl.pallas_call(
        _decode_kernel,
        out_shape=jax.ShapeDtypeStruct((8, 128), jnp.float32),
        in_specs=[pl.BlockSpec((1, 8, 8, 128), lambda: (0, 0, 0, 0))],
        out_specs=pl.BlockSpec((8, 128), lambda: (0, 0)),
    )(x)
    return out[:1, :7]


# speedup vs baseline: 3.0229x; 3.0229x over previous
import jax
import jax.numpy as jnp
from jax.experimental import pallas as pl

_IMG_H = 100.0
_IMG_W = 200.0


def _decode_kernel(x_ref, o_ref):
    cx = x_ref[0, 0, 0, 0]
    cy = x_ref[0, 1, 0, 0]
    w = x_ref[0, 2, 0, 0]
    h = x_ref[0, 3, 0, 0]
    s = x_ref[0, 4, 0, 0]
    dw = w * 0.5
    dh = h * 0.5
    x1 = jnp.clip(cx - dw, 0.0, _IMG_W)
    y1 = jnp.clip(cy - dh, 0.0, _IMG_H)
    x2 = jnp.clip(cx + dw, 0.0, _IMG_W)
    y2 = jnp.clip(cy + dh, 0.0, _IMG_H)
    col = jax.lax.broadcasted_iota(jnp.int32, (8, 128), 1)
    row = jnp.zeros((8, 128), jnp.float32)
    row = jnp.where(col == 1, x1, row)
    row = jnp.where(col == 2, y1, row)
    row = jnp.where(col == 3, x2, row)
    row = jnp.where(col == 4, y2, row)
    row = jnp.where(col == 5, s, row)
    o_ref[...] = row


def kernel(x):
    out = pl.pallas_call(
        _decode_kernel,
        out_shape=jax.ShapeDtypeStruct((8, 128), jnp.float32),
        grid=(1,),
        in_specs=[pl.BlockSpec((1, 8, 8, 128), lambda i: (0, 0, 0, 0))],
        out_specs=pl.BlockSpec((8, 128), lambda i: (0, 0)),
    )(x)
    return out[:1, :7]


# speedup vs baseline: 3.1156x; 1.0307x over previous
import jax
import jax.numpy as jnp
from jax.experimental import pallas as pl

_IMG_H = 100.0
_IMG_W = 200.0


def _decode_kernel(x_ref, o_ref):
    cx = x_ref[0, 0, 0, 0]
    cy = x_ref[0, 1, 0, 0]
    w = x_ref[0, 2, 0, 0]
    h = x_ref[0, 3, 0, 0]
    s = x_ref[0, 4, 0, 0]
    dw = w * 0.5
    dh = h * 0.5
    x1 = jnp.clip(cx - dw, 0.0, _IMG_W)
    y1 = jnp.clip(cy - dh, 0.0, _IMG_H)
    x2 = jnp.clip(cx + dw, 0.0, _IMG_W)
    y2 = jnp.clip(cy + dh, 0.0, _IMG_H)
    col = jax.lax.broadcasted_iota(jnp.int32, (1, 7), 1)
    row = jnp.zeros((1, 7), jnp.float32)
    row = jnp.where(col == 1, x1, row)
    row = jnp.where(col == 2, y1, row)
    row = jnp.where(col == 3, x2, row)
    row = jnp.where(col == 4, y2, row)
    row = jnp.where(col == 5, s, row)
    o_ref[...] = row


def kernel(x):
    return pl.pallas_call(
        _decode_kernel,
        out_shape=jax.ShapeDtypeStruct((1, 7), jnp.float32),
        grid=(1,),
        in_specs=[pl.BlockSpec((1, 8, 8, 128), lambda i: (0, 0, 0, 0))],
        out_specs=pl.BlockSpec((1, 7), lambda i: (0, 0)),
    )(x)
